# baseline (speedup 1.0000x reference)
import jax
import jax.numpy as jnp
from jax import lax
from jax.experimental import pallas as pl
from jax.experimental.pallas import tpu as pltpu
from jax.experimental.pallas import tpu_sc as plsc

N = 10000
E = 320000
D = 128
NT = 32
CHUNK = 64
NCHUNK = 160
GRP = 8
NGRP = NCHUNK // GRP
EPT = CHUNK * NCHUNK
EP = NT * EPT
RPT = 632
NP = 16 * RPT

MBLK = 1000


_SC_MESH = plsc.VectorSubcoreMesh(core_axis_name="c", subcore_axis_name="s")


def _fill_rows(ref, val):
  v = jnp.full((16,), val, jnp.float32)

  def fill(r, carry):
    for cc in range(D // 16):
      ref[r, pl.ds(cc * 16, 16)] = v
    return carry
  lax.fori_loop(0, CHUNK, fill, 0)


def _zero_slice(zsrc, acc_sh, r0):
  for k in range(RPT // CHUNK):
    pltpu.sync_copy(zsrc, acc_sh.at[pl.ds(r0 + CHUNK * k, CHUNK)])
  rem = RPT % CHUNK
  pltpu.sync_copy(zsrc.at[pl.ds(0, rem)], acc_sh.at[pl.ds(r0 + RPT - rem, rem)])


NB = 5
DG = 4


def _make_sc_agg():
  scratch = [
      pltpu.VMEM((GRP, CHUNK), jnp.int32),
      pltpu.VMEM((GRP, CHUNK), jnp.int32),
      pltpu.VMEM((GRP, CHUNK), jnp.int32),
      pltpu.VMEM((GRP, CHUNK), jnp.int32),
  ] + [pltpu.VMEM((CHUNK, D), jnp.float32) for _ in range(NB)] + [
      pltpu.SemaphoreType.DMA for _ in range(2 * NB)
  ] + [
      pltpu.SemaphoreType.DMA,
      pltpu.SemaphoreType.DMA,
      pltpu.VMEM_SHARED((NP, D), jnp.float32),
  ]

  def body(y_hbm, src_hbm, dst_hbm, acc_out, *rest):
    src_a, dst_a, src_b, dst_b = rest[:4]
    rows = rest[4:4 + NB]
    semg = rest[4 + NB:4 + 2 * NB]
    sems = rest[4 + 2 * NB:4 + 3 * NB]
    sem_a, sem_b, acc_sh = rest[4 + 3 * NB:]

    c = lax.axis_index("c")
    s = lax.axis_index("s")
    wid = c * 16 + s

    def idx_start(g, sbuf, dbuf, sem):
      pltpu.async_copy(src_hbm.at[wid, pl.ds(g * GRP, GRP)], sbuf, sem)
      pltpu.async_copy(dst_hbm.at[wid, pl.ds(g * GRP, GRP)], dbuf, sem)

    def idx_wait(sbuf, dbuf, sem):
      pltpu.make_async_copy(src_hbm.at[wid, pl.ds(0, GRP)], sbuf, sem).wait()
      pltpu.make_async_copy(dst_hbm.at[wid, pl.ds(0, GRP)], dbuf, sem).wait()

    idx_start(0, src_a, dst_a, sem_a)
    idx_start(1, src_b, dst_b, sem_b)

    _fill_rows(rows[0], 0.0)
    r0 = s * RPT
    _zero_slice(rows[0], acc_sh, r0)
    plsc.subcore_barrier()

    def outer(i, carry):
      idx_wait(src_a, dst_a, sem_a)
      g, sd = {}, {}

      def dref(j):
        return (dst_a if j < 8 else dst_b).at[j % 8]

      for j in range(16):
        if j == 0:
          @pl.when(i > 0)
          def _():
            idx_start(2 * i + 1, src_b, dst_b, sem_b)
        if j == 8:
          idx_wait(src_b, dst_b, sem_b)
        b = j % NB
        if j >= NB:
          sd[j - NB].wait()
        sbuf = src_a if j < 8 else src_b
        g[j] = pltpu.async_copy(y_hbm.at[sbuf.at[j % 8]], rows[b], semg[b])
        if j >= DG:
          g[j - DG].wait()
          sd[j - DG] = pltpu.async_copy(rows[(j - DG) % NB],
                                        acc_sh.at[dref(j - DG)],
                                        sems[(j - DG) % NB], add=True)
        if j == 12:
          @pl.when(i < NGRP // 2 - 1)
          def _():
            idx_start(2 * i + 2, src_a, dst_a, sem_a)
      for j in (12, 13, 14, 15):
        g[j].wait()
        sd[j] = pltpu.async_copy(rows[j % NB], acc_sh.at[dref(j)],
                                 sems[j % NB], add=True)
      for j in (11, 12, 13, 14, 15):
        sd[j].wait()
      return carry
    lax.fori_loop(0, NGRP // 2, outer, 0)

    plsc.subcore_barrier()
    pltpu.sync_copy(acc_sh.at[pl.ds(r0, RPT)], acc_out.at[c, pl.ds(r0, RPT)])

  return pl.kernel(
      body, out_type=[jax.ShapeDtypeStruct((2, NP, D), jnp.float32)],
      mesh=_SC_MESH, scratch_types=scratch)


def _make_sc_cnt():
  scratch = [
      pltpu.VMEM((GRP, CHUNK), jnp.int32),
      pltpu.VMEM((GRP, CHUNK), jnp.int32),
      pltpu.VMEM((CHUNK, D), jnp.float32),
      pltpu.SemaphoreType.DMA,
      pltpu.SemaphoreType.DMA,
      pltpu.VMEM_SHARED((NP, D), jnp.float32),
  ]

  def body(dst_hbm, cnt_out, dst_a, dst_b, ones_v, sem_a, sem_b, cnt_sh):
    c = lax.axis_index("c")
    s = lax.axis_index("s")
    wid = c * 16 + s

    def idx_start(g, dbuf, sem):
      pltpu.async_copy(dst_hbm.at[wid, pl.ds(g * GRP, GRP)], dbuf, sem)

    def idx_wait(dbuf, sem):
      pltpu.make_async_copy(dst_hbm.at[wid, pl.ds(0, GRP)], dbuf, sem).wait()

    idx_start(0, dst_a, sem_a)
    idx_start(1, dst_b, sem_b)

    _fill_rows(ones_v, 0.0)
    r0 = s * RPT
    _zero_slice(ones_v, cnt_sh, r0)
    _fill_rows(ones_v, 1.0)
    plsc.subcore_barrier()

    def process_group(dbuf):
      for j in range(GRP):
        pltpu.sync_copy(ones_v, cnt_sh.at[dbuf.at[j]], add=True)

    def outer(i, carry):
      idx_wait(dst_a, sem_a)
      process_group(dst_a)

      @pl.when(i < NGRP // 2 - 1)
      def _():
        idx_start(2 * i + 2, dst_a, sem_a)

      idx_wait(dst_b, sem_b)
      process_group(dst_b)

      @pl.when(i < NGRP // 2 - 1)
      def _():
        idx_start(2 * i + 3, dst_b, sem_b)
      return carry
    lax.fori_loop(0, NGRP // 2, outer, 0)

    plsc.subcore_barrier()
    pltpu.sync_copy(cnt_sh.at[pl.ds(r0, RPT)], cnt_out.at[c, pl.ds(r0, RPT)])

  return pl.kernel(
      body, out_type=[jax.ShapeDtypeStruct((2, NP, D), jnp.float32)],
      mesh=_SC_MESH, scratch_types=scratch)


_sc_agg = _make_sc_agg()
_sc_cnt = _make_sc_cnt()


def _row_spec(w):
  return pl.BlockSpec((MBLK, w), lambda i: (i, 0))


def _acc_spec(core):
  return pl.BlockSpec((1, MBLK, D), lambda i: (core, i, 0))


def _full_spec(h, w):
  return pl.BlockSpec((h, w), lambda i: (0, 0))


def _dense_in_body(x_ref, wl_ref, wr_ref, b_ref, y_ref, z_ref):
  x = x_ref[...]
  y_ref[...] = jnp.dot(x, wl_ref[...], preferred_element_type=jnp.float32)
  z_ref[...] = jnp.dot(x, wr_ref[...],
                       preferred_element_type=jnp.float32) + b_ref[...]


def _dense_mid_body(a0_ref, a1_ref, c0_ref, c1_ref, z1_ref,
                    wl_ref, wr_ref, b_ref, y_ref, z_ref):
  cnt = c0_ref[0, :, 0:1] + c1_ref[0, :, 0:1]
  inv = 1.0 / jnp.maximum(cnt, 1.0)
  h = jnp.maximum((a0_ref[0] + a1_ref[0]) * inv + z1_ref[...], 0.0)
  y_ref[...] = jnp.dot(h, wl_ref[...], preferred_element_type=jnp.float32)
  z_ref[...] = jnp.dot(h, wr_ref[...],
                       preferred_element_type=jnp.float32) + b_ref[...]


def _combine_body(a0_ref, a1_ref, c0_ref, c1_ref, z2_ref, o_ref):
  cnt = c0_ref[0, :, 0:1] + c1_ref[0, :, 0:1]
  inv = 1.0 / jnp.maximum(cnt, 1.0)
  o_ref[...] = (a0_ref[0] + a1_ref[0]) * inv + z2_ref[...]


def _dense_in(x, wl, wr, b):
  return pl.pallas_call(
      _dense_in_body,
      grid=(N // MBLK,),
      in_specs=[_row_spec(D), _full_spec(D, D), _full_spec(D, D),
                _full_spec(1, D)],
      out_specs=[_row_spec(D), _row_spec(D)],
      out_shape=[jax.ShapeDtypeStruct((N, D), jnp.float32)] * 2,
  )(x, wl, wr, b)


def _dense_mid(acc, cnt, z1, wl, wr, b):
  return pl.pallas_call(
      _dense_mid_body,
      grid=(N // MBLK,),
      in_specs=[_acc_spec(0), _acc_spec(1), _acc_spec(0), _acc_spec(1),
                _row_spec(D), _full_spec(D, D), _full_spec(D, D),
                _full_spec(1, D)],
      out_specs=[_row_spec(D), _row_spec(D)],
      out_shape=[jax.ShapeDtypeStruct((N, D), jnp.float32)] * 2,
  )(acc, acc, cnt, cnt, z1, wl, wr, b)


def _combine(acc, cnt, z2):
  return pl.pallas_call(
      _combine_body,
      grid=(N // MBLK,),
      in_specs=[_acc_spec(0), _acc_spec(1), _acc_spec(0), _acc_spec(1),
                _row_spec(D)],
      out_specs=_row_spec(D),
      out_shape=jax.ShapeDtypeStruct((N, D), jnp.float32),
  )(acc, acc, cnt, cnt, z2)


def kernel(x, edge_index, W_l1, W_r1, b1, W_l2, W_r2, b2):
  src = edge_index[0].astype(jnp.int32)
  dst = edge_index[1].astype(jnp.int32)

  tile_ids = (jnp.arange(EP, dtype=jnp.int32) // EPT) % 16
  src_p = jnp.zeros((EP,), jnp.int32).at[:E].set(src).reshape(NT, NCHUNK, CHUNK)
  dst_p = (N + tile_ids).at[:E].set(dst).reshape(NT, NCHUNK, CHUNK)

  b1r = b1.reshape(1, D)
  b2r = b2.reshape(1, D)

  y1, z1 = _dense_in(x, W_l1, W_r1, b1r)
  cnt = _sc_cnt(dst_p)[0]
  acc1 = _sc_agg(y1, src_p, dst_p)[0]
  y2, z2 = _dense_mid(acc1, cnt, z1, W_l2, W_r2, b2r)
  acc2 = _sc_agg(y2, src_p, dst_p)[0]
  return _combine(acc2, cnt, z2)


# speedup vs baseline: 1.0838x; 1.0838x over previous
import jax
import jax.numpy as jnp
from jax import lax
from jax.experimental import pallas as pl
from jax.experimental.pallas import tpu as pltpu
from jax.experimental.pallas import tpu_sc as plsc

N = 10000
E = 320000
D = 128
NT = 32
CHUNK = 128
NCHUNK = 80
GRP = 8
NGRP = NCHUNK // GRP
EPT = CHUNK * NCHUNK
EP = NT * EPT
RPT = 632
NP = 16 * RPT

MBLK = 1000


_SC_MESH = plsc.VectorSubcoreMesh(core_axis_name="c", subcore_axis_name="s")


def _fill_rows(ref, val):
  v = jnp.full((16,), val, jnp.float32)

  def fill(r, carry):
    for cc in range(D // 16):
      ref[r, pl.ds(cc * 16, 16)] = v
    return carry
  lax.fori_loop(0, CHUNK, fill, 0)


def _zero_slice(zsrc, acc_sh, r0):
  for k in range(RPT // CHUNK):
    pltpu.sync_copy(zsrc, acc_sh.at[pl.ds(r0 + CHUNK * k, CHUNK)])
  rem = RPT % CHUNK
  pltpu.sync_copy(zsrc.at[pl.ds(0, rem)], acc_sh.at[pl.ds(r0 + RPT - rem, rem)])


NB = 2
DG = 1


def _make_sc_agg():
  scratch = [
      pltpu.VMEM((GRP, CHUNK), jnp.int32),
      pltpu.VMEM((GRP, CHUNK), jnp.int32),
      pltpu.VMEM((GRP, CHUNK), jnp.int32),
      pltpu.VMEM((GRP, CHUNK), jnp.int32),
  ] + [pltpu.VMEM((CHUNK, D), jnp.float32) for _ in range(NB)] + [
      pltpu.SemaphoreType.DMA for _ in range(2 * NB)
  ] + [
      pltpu.SemaphoreType.DMA,
      pltpu.SemaphoreType.DMA,
      pltpu.VMEM_SHARED((NP, D), jnp.float32),
  ]

  def body(y_hbm, src_hbm, dst_hbm, acc_out, *rest):
    src_a, dst_a, src_b, dst_b = rest[:4]
    rows = rest[4:4 + NB]
    semg = rest[4 + NB:4 + 2 * NB]
    sems = rest[4 + 2 * NB:4 + 3 * NB]
    sem_a, sem_b, acc_sh = rest[4 + 3 * NB:]

    c = lax.axis_index("c")
    s = lax.axis_index("s")
    wid = c * 16 + s

    def idx_start(g, sbuf, dbuf, sem):
      pltpu.async_copy(src_hbm.at[wid, pl.ds(g * GRP, GRP)], sbuf, sem)
      pltpu.async_copy(dst_hbm.at[wid, pl.ds(g * GRP, GRP)], dbuf, sem)

    def idx_wait(sbuf, dbuf, sem):
      pltpu.make_async_copy(src_hbm.at[wid, pl.ds(0, GRP)], sbuf, sem).wait()
      pltpu.make_async_copy(dst_hbm.at[wid, pl.ds(0, GRP)], dbuf, sem).wait()

    idx_start(0, src_a, dst_a, sem_a)
    idx_start(1, src_b, dst_b, sem_b)

    _fill_rows(rows[0], 0.0)
    r0 = s * RPT
    _zero_slice(rows[0], acc_sh, r0)
    plsc.subcore_barrier()

    def outer(i, carry):
      idx_wait(src_a, dst_a, sem_a)
      g, sd = {}, {}

      def dref(j):
        return (dst_a if j < 8 else dst_b).at[j % 8]

      for j in range(16):
        if j == 0:
          @pl.when(i > 0)
          def _():
            idx_start(2 * i + 1, src_b, dst_b, sem_b)
        if j == 8:
          idx_wait(src_b, dst_b, sem_b)
        b = j % NB
        if j >= NB:
          sd[j - NB].wait()
        sbuf = src_a if j < 8 else src_b
        g[j] = pltpu.async_copy(y_hbm.at[sbuf.at[j % 8]], rows[b], semg[b])
        if j >= DG:
          g[j - DG].wait()
          sd[j - DG] = pltpu.async_copy(rows[(j - DG) % NB],
                                        acc_sh.at[dref(j - DG)],
                                        sems[(j - DG) % NB], add=True)
        if j == 10:
          @pl.when(i < NGRP // 2 - 1)
          def _():
            idx_start(2 * i + 2, src_a, dst_a, sem_a)
      g[15].wait()
      sd[15] = pltpu.async_copy(rows[15 % NB], acc_sh.at[dref(15)],
                                sems[15 % NB], add=True)
      sd[14].wait()
      sd[15].wait()
      return carry
    lax.fori_loop(0, NGRP // 2, outer, 0)

    plsc.subcore_barrier()
    pltpu.sync_copy(acc_sh.at[pl.ds(r0, RPT)], acc_out.at[c, pl.ds(r0, RPT)])

  return pl.kernel(
      body, out_type=[jax.ShapeDtypeStruct((2, NP, D), jnp.float32)],
      mesh=_SC_MESH, scratch_types=scratch)


def _make_sc_cnt():
  scratch = [
      pltpu.VMEM((GRP, CHUNK), jnp.int32),
      pltpu.VMEM((GRP, CHUNK), jnp.int32),
      pltpu.VMEM((CHUNK, D), jnp.float32),
      pltpu.SemaphoreType.DMA,
      pltpu.SemaphoreType.DMA,
      pltpu.VMEM_SHARED((NP, D), jnp.float32),
  ]

  def body(dst_hbm, cnt_out, dst_a, dst_b, ones_v, sem_a, sem_b, cnt_sh):
    c = lax.axis_index("c")
    s = lax.axis_index("s")
    wid = c * 16 + s

    def idx_start(g, dbuf, sem):
      pltpu.async_copy(dst_hbm.at[wid, pl.ds(g * GRP, GRP)], dbuf, sem)

    def idx_wait(dbuf, sem):
      pltpu.make_async_copy(dst_hbm.at[wid, pl.ds(0, GRP)], dbuf, sem).wait()

    idx_start(0, dst_a, sem_a)
    idx_start(1, dst_b, sem_b)

    _fill_rows(ones_v, 0.0)
    r0 = s * RPT
    _zero_slice(ones_v, cnt_sh, r0)
    _fill_rows(ones_v, 1.0)
    plsc.subcore_barrier()

    def process_group(dbuf):
      for j in range(GRP):
        pltpu.sync_copy(ones_v, cnt_sh.at[dbuf.at[j]], add=True)

    def outer(i, carry):
      idx_wait(dst_a, sem_a)
      process_group(dst_a)

      @pl.when(i < NGRP // 2 - 1)
      def _():
        idx_start(2 * i + 2, dst_a, sem_a)

      idx_wait(dst_b, sem_b)
      process_group(dst_b)

      @pl.when(i < NGRP // 2 - 1)
      def _():
        idx_start(2 * i + 3, dst_b, sem_b)
      return carry
    lax.fori_loop(0, NGRP // 2, outer, 0)

    plsc.subcore_barrier()
    pltpu.sync_copy(cnt_sh.at[pl.ds(r0, RPT)], cnt_out.at[c, pl.ds(r0, RPT)])

  return pl.kernel(
      body, out_type=[jax.ShapeDtypeStruct((2, NP, D), jnp.float32)],
      mesh=_SC_MESH, scratch_types=scratch)


_sc_agg = _make_sc_agg()
_sc_cnt = _make_sc_cnt()


def _row_spec(w):
  return pl.BlockSpec((MBLK, w), lambda i: (i, 0))


def _acc_spec(core):
  return pl.BlockSpec((1, MBLK, D), lambda i: (core, i, 0))


def _full_spec(h, w):
  return pl.BlockSpec((h, w), lambda i: (0, 0))


def _dense_in_body(x_ref, wl_ref, wr_ref, b_ref, y_ref, z_ref):
  x = x_ref[...]
  y_ref[...] = jnp.dot(x, wl_ref[...], preferred_element_type=jnp.float32)
  z_ref[...] = jnp.dot(x, wr_ref[...],
                       preferred_element_type=jnp.float32) + b_ref[...]


def _dense_mid_body(a0_ref, a1_ref, c0_ref, c1_ref, z1_ref,
                    wl_ref, wr_ref, b_ref, y_ref, z_ref):
  cnt = c0_ref[0, :, 0:1] + c1_ref[0, :, 0:1]
  inv = 1.0 / jnp.maximum(cnt, 1.0)
  h = jnp.maximum((a0_ref[0] + a1_ref[0]) * inv + z1_ref[...], 0.0)
  y_ref[...] = jnp.dot(h, wl_ref[...], preferred_element_type=jnp.float32)
  z_ref[...] = jnp.dot(h, wr_ref[...],
                       preferred_element_type=jnp.float32) + b_ref[...]


def _combine_body(a0_ref, a1_ref, c0_ref, c1_ref, z2_ref, o_ref):
  cnt = c0_ref[0, :, 0:1] + c1_ref[0, :, 0:1]
  inv = 1.0 / jnp.maximum(cnt, 1.0)
  o_ref[...] = (a0_ref[0] + a1_ref[0]) * inv + z2_ref[...]


def _dense_in(x, wl, wr, b):
  return pl.pallas_call(
      _dense_in_body,
      grid=(N // MBLK,),
      in_specs=[_row_spec(D), _full_spec(D, D), _full_spec(D, D),
                _full_spec(1, D)],
      out_specs=[_row_spec(D), _row_spec(D)],
      out_shape=[jax.ShapeDtypeStruct((N, D), jnp.float32)] * 2,
  )(x, wl, wr, b)


def _dense_mid(acc, cnt, z1, wl, wr, b):
  return pl.pallas_call(
      _dense_mid_body,
      grid=(N // MBLK,),
      in_specs=[_acc_spec(0), _acc_spec(1), _acc_spec(0), _acc_spec(1),
                _row_spec(D), _full_spec(D, D), _full_spec(D, D),
                _full_spec(1, D)],
      out_specs=[_row_spec(D), _row_spec(D)],
      out_shape=[jax.ShapeDtypeStruct((N, D), jnp.float32)] * 2,
  )(acc, acc, cnt, cnt, z1, wl, wr, b)


def _combine(acc, cnt, z2):
  return pl.pallas_call(
      _combine_body,
      grid=(N // MBLK,),
      in_specs=[_acc_spec(0), _acc_spec(1), _acc_spec(0), _acc_spec(1),
                _row_spec(D)],
      out_specs=_row_spec(D),
      out_shape=jax.ShapeDtypeStruct((N, D), jnp.float32),
  )(acc, acc, cnt, cnt, z2)


def kernel(x, edge_index, W_l1, W_r1, b1, W_l2, W_r2, b2):
  src = edge_index[0].astype(jnp.int32)
  dst = edge_index[1].astype(jnp.int32)

  tile_ids = (jnp.arange(EP, dtype=jnp.int32) // EPT) % 16
  src_p = jnp.zeros((EP,), jnp.int32).at[:E].set(src).reshape(NT, NCHUNK, CHUNK)
  dst_p = (N + tile_ids).at[:E].set(dst).reshape(NT, NCHUNK, CHUNK)

  b1r = b1.reshape(1, D)
  b2r = b2.reshape(1, D)

  y1, z1 = _dense_in(x, W_l1, W_r1, b1r)
  cnt = _sc_cnt(dst_p)[0]
  acc1 = _sc_agg(y1, src_p, dst_p)[0]
  y2, z2 = _dense_mid(acc1, cnt, z1, W_l2, W_r2, b2r)
  acc2 = _sc_agg(y2, src_p, dst_p)[0]
  return _combine(acc2, cnt, z2)
